# baseline (speedup 1.0000x reference)
import jax
import jax.numpy as jnp
from jax.experimental import pallas as pl

N = 10000
E = 160000
D_IN = 500
HID = 1000
HEADS = 8
D_OUT = 200
B = 1024


def _mm_kernel(x_ref, w_ref, o_ref):
    o_ref[...] = jnp.dot(x_ref[...], w_ref[...], preferred_element_type=jnp.float32)


def _matmul(x, w, bm):
    m, k = x.shape
    _, n = w.shape
    return pl.pallas_call(
        _mm_kernel,
        grid=(m // bm,),
        in_specs=[
            pl.BlockSpec((bm, k), lambda i: (i, 0)),
            pl.BlockSpec((k, n), lambda i: (0, 0)),
        ],
        out_specs=pl.BlockSpec((bm, n), lambda i: (i, 0)),
        out_shape=jax.ShapeDtypeStruct((m, n), jnp.float32),
    )(x, w)


def _gat(x, src, dst, W, a_s, a_d, b, heads, ch, concat, bm):
    xt = _matmul(x, W, bm).reshape(-1, heads, ch)
    a_src = jnp.sum(xt * a_s[None, :, :], axis=-1)
    a_dst = jnp.sum(xt * a_d[None, :, :], axis=-1)
    e = a_src[src] + a_dst[dst]
    e = jnp.where(e > 0, e, 0.2 * e)
    m = jax.ops.segment_max(e, dst, num_segments=N)
    m = jnp.where(jnp.isfinite(m), m, 0.0)
    ex = jnp.exp(e - m[dst])
    s = jax.ops.segment_sum(ex, dst, num_segments=N)
    alpha = ex / (s[dst] + 1e-16)
    outs = []
    for h in range(heads):
        msg = alpha[:, h][:, None] * xt[src, h, :]
        outs.append(jax.ops.segment_sum(msg, dst, num_segments=N))
    out = jnp.stack(outs, axis=1)
    out = out.reshape(N, heads * ch) if concat else jnp.mean(out, axis=1)
    return out + b


def kernel(x1_id, x2_id, edge_index, x, W1, att_src1, att_dst1, b1, W2,
           att_src2, att_dst2, b2, Wl1, bl1, Wl2, bl2, Wf1, bf1, g1, be1,
           rm1, rv1, Wf2, bf2, g2, be2, rm2, rv2, Wf3, bf3):
    loops = jnp.arange(N, dtype=edge_index.dtype)
    src = jnp.concatenate([edge_index[0], loops])
    dst = jnp.concatenate([edge_index[1], loops])
    h = _gat(x, src, dst, W1, att_src1, att_dst1, b1, HEADS, HID, True, 400)
    h = jnp.where(h > 0, h, jnp.exp(h) - 1.0)
    h = _gat(h, src, dst, W2, att_src2, att_dst2, b2, 1, D_OUT, False, 400)
    z1 = h[x1_id] @ Wl1 + bl1
    z2 = h[x2_id] @ Wl2 + bl2
    z = jnp.concatenate([z1, z2], axis=1)
    z = jnp.maximum(z @ Wf1 + bf1, 0.0)
    z = (z - rm1) / jnp.sqrt(rv1 + 1e-5) * g1 + be1
    z = jnp.maximum(z @ Wf2 + bf2, 0.0)
    z = (z - rm2) / jnp.sqrt(rv2 + 1e-5) * g2 + be2
    z = z @ Wf3 + bf3
    return jax.nn.softmax(z, axis=1)
